# Initial kernel scaffold; baseline (speedup 1.0000x reference)
import jax
import jax.numpy as jnp
from jax.experimental import pallas as pl


def kernel(ray_samples_packed, rgb_samples, weights_samples):
    raise NotImplementedError("write your pallas kernel here")


# speedup vs baseline: 1.9644x; 1.9644x over previous
import functools

import jax
import jax.numpy as jnp
from jax import lax
from jax.experimental import pallas as pl
from jax.experimental.pallas import tpu as pltpu
from jax.experimental.pallas import tpu_sc as plsc

N = 4194304
R = 65536
NC = 2
NS = 16
W = NC * NS
C = N // W
CH = 2048
NCH = C // CH
NG = CH // 16
RT3 = 3 * R // NS


def _sc_body(ids_hbm, rgb_hbm, w_hbm, zeros_hbm, out_hbm,
             ids_v, rgb_v, w_v, con_v, tgt_v, acc):
    cid = lax.axis_index("c")
    sid = lax.axis_index("s")
    wid = cid * NS + sid

    pltpu.sync_copy(zeros_hbm.at[pl.ds(sid * RT3, RT3)],
                    acc.at[pl.ds(sid * RT3, RT3)])
    plsc.subcore_barrier()

    ii = lax.iota(jnp.int32, 16)
    q = [lax.shift_right_logical((16 * v + ii) * 21846, 16) for v in range(3)]
    col = [(16 * v + ii) - 3 * q[v] for v in range(3)]

    def chunk(ch, _):
        base = wid * C + ch * CH
        pltpu.sync_copy(ids_hbm.at[pl.ds(base, CH)], ids_v)
        pltpu.sync_copy(rgb_hbm.at[pl.ds(3 * base, 3 * CH)], rgb_v)
        pltpu.sync_copy(w_hbm.at[pl.ds(base, CH)], w_v)

        def subchunk(j, _):
            for gg in range(8):
                s0 = 128 * j + 16 * gg
                for v in range(3):
                    off = 48 * gg + 16 * v
                    rr, cc = off // 128, off % 128
                    sq = s0 + q[v]
                    id16 = plsc.load_gather(ids_v, [sq])
                    w16 = plsc.load_gather(w_v, [sq])
                    rgb16 = rgb_v[pl.ds(384 * j + off, 16)]
                    con_v[3 * j + rr, pl.ds(cc, 16)] = rgb16 * w16
                    tgt_v[3 * j + rr, pl.ds(cc, 16)] = id16 * 3 + col[v]
            return 0

        lax.fori_loop(0, CH // 128, subchunk, 0)
        for r in range(3 * CH // 128):
            pltpu.sync_copy(con_v.at[r], acc.at[tgt_v.at[r]], add=True)
        return 0

    lax.fori_loop(0, NCH, chunk, 0)
    plsc.subcore_barrier()

    pltpu.sync_copy(acc.at[pl.ds(sid * RT3, RT3)],
                    out_hbm.at[cid].at[pl.ds(sid * RT3, RT3)])


_sc_scatter = functools.partial(
    pl.kernel,
    mesh=plsc.VectorSubcoreMesh(core_axis_name="c", subcore_axis_name="s",
                                num_cores=NC, num_subcores=NS),
    compiler_params=pltpu.CompilerParams(needs_layout_passes=False),
    out_type=jax.ShapeDtypeStruct((NC, 3 * R), jnp.float32),
    scratch_types=[
        pltpu.VMEM((CH,), jnp.int32),
        pltpu.VMEM((3 * CH,), jnp.float32),
        pltpu.VMEM((CH,), jnp.float32),
        pltpu.VMEM((3 * CH // 128, 128), jnp.float32),
        pltpu.VMEM((3 * CH // 128, 128), jnp.int32),
        pltpu.VMEM_SHARED((3 * R,), jnp.float32),
    ],
)(_sc_body)


def _merge_body(p_ref, o_ref):
    o_ref[...] = p_ref[0] + p_ref[1]


def kernel(ray_samples_packed, rgb_samples, weights_samples):
    ids = ray_samples_packed.astype(jnp.int32)
    rgb_flat = rgb_samples.reshape(-1)
    w_flat = weights_samples.reshape(-1)
    zeros = jnp.zeros((3 * R,), jnp.float32)
    partial = _sc_scatter(ids, rgb_flat, w_flat, zeros)
    out = pl.pallas_call(
        _merge_body,
        out_shape=jax.ShapeDtypeStruct((R * 3 // 128, 128), jnp.float32),
    )(partial.reshape(NC, R * 3 // 128, 128))
    return out.reshape(R, 3)


# speedup vs baseline: 2.0934x; 1.0657x over previous
import functools

import jax
import jax.numpy as jnp
from jax import lax
from jax.experimental import pallas as pl
from jax.experimental.pallas import tpu as pltpu
from jax.experimental.pallas import tpu_sc as plsc

N = 4194304
R = 65536
NC = 2
NS = 16
W = NC * NS
C = N // W
CH = 2048
NCH = C // CH
NR = 3 * CH // 128
RT3 = 3 * R // NS


def _sc_body(ids_hbm, rgb_hbm, w_hbm, zeros_hbm, out_hbm,
             ids_v, rgb_v, w_v, con_v, tgt_v, acc,
             sem_in, sem_sc, sem_z):
    cid = lax.axis_index("c")
    sid = lax.axis_index("s")
    wid = cid * NS + sid

    pltpu.async_copy(zeros_hbm.at[pl.ds(sid * RT3, RT3)],
                     acc.at[pl.ds(sid * RT3, RT3)], sem_z).wait()
    plsc.subcore_barrier()

    ii = lax.iota(jnp.int32, 16)
    q = [lax.shift_right_logical((16 * v + ii) * 21846, 16) for v in range(3)]
    col = [(16 * v + ii) - 3 * q[v] for v in range(3)]

    def in_copies(p, ch):
        base = wid * C + ch * CH
        return (
            pltpu.make_async_copy(ids_hbm.at[pl.ds(base, CH)],
                                  ids_v[p], sem_in[p]),
            pltpu.make_async_copy(rgb_hbm.at[pl.ds(3 * base, 3 * CH)],
                                  rgb_v[p], sem_in[p]),
            pltpu.make_async_copy(w_hbm.at[pl.ds(base, CH)],
                                  w_v[p], sem_in[p]),
        )

    def issue_in(p, ch):
        for c in in_copies(p, ch):
            c.start()

    def wait_in(p, ch):
        for c in in_copies(p, ch):
            c.wait()

    def compute(p):
        def subchunk(j, _):
            for gg in range(8):
                s0 = 128 * j + 16 * gg
                for v in range(3):
                    off = 48 * gg + 16 * v
                    rr, cc = off // 128, off % 128
                    sq = s0 + q[v]
                    id16 = plsc.load_gather(ids_v[p], [sq])
                    w16 = plsc.load_gather(w_v[p], [sq])
                    rgb16 = rgb_v[p][pl.ds(384 * j + off, 16)]
                    con_v[p][3 * j + rr, pl.ds(cc, 16)] = rgb16 * w16
                    tgt_v[p][3 * j + rr, pl.ds(cc, 16)] = id16 * 3 + col[v]
            return 0

        lax.fori_loop(0, CH // 128, subchunk, 0)

    def fire_scatters(p):
        for r in range(NR):
            pltpu.async_copy(con_v[p].at[r], acc.at[tgt_v[p].at[r]],
                             sem_sc[p], add=True)

    def drain_scatters(p):
        pltpu.make_async_copy(rgb_hbm.at[pl.ds(0, NR * 128)],
                              rgb_v[p], sem_sc[p]).wait()

    issue_in(0, 0)

    def pair(ch2, _):
        ch0 = 2 * ch2

        @pl.when(ch2 > 0)
        def _():
            drain_scatters(0)
        issue_in(1, ch0 + 1)
        wait_in(0, ch0)
        compute(0)
        fire_scatters(0)

        @pl.when(ch2 > 0)
        def _():
            drain_scatters(1)

        @pl.when(ch2 < NCH // 2 - 1)
        def _():
            issue_in(0, ch0 + 2)
        wait_in(1, ch0 + 1)
        compute(1)
        fire_scatters(1)
        return 0

    lax.fori_loop(0, NCH // 2, pair, 0)
    drain_scatters(0)
    drain_scatters(1)
    plsc.subcore_barrier()

    pltpu.async_copy(acc.at[pl.ds(sid * RT3, RT3)],
                     out_hbm.at[cid].at[pl.ds(sid * RT3, RT3)], sem_z).wait()


_sc_scatter = functools.partial(
    pl.kernel,
    mesh=plsc.VectorSubcoreMesh(core_axis_name="c", subcore_axis_name="s",
                                num_cores=NC, num_subcores=NS),
    compiler_params=pltpu.CompilerParams(needs_layout_passes=False),
    out_type=jax.ShapeDtypeStruct((NC, 3 * R), jnp.float32),
    scratch_types=[
        [pltpu.VMEM((CH,), jnp.int32)] * 2,
        [pltpu.VMEM((3 * CH,), jnp.float32)] * 2,
        [pltpu.VMEM((CH,), jnp.float32)] * 2,
        [pltpu.VMEM((NR, 128), jnp.float32)] * 2,
        [pltpu.VMEM((NR, 128), jnp.int32)] * 2,
        pltpu.VMEM_SHARED((3 * R,), jnp.float32),
        [pltpu.SemaphoreType.DMA] * 2,
        [pltpu.SemaphoreType.DMA] * 2,
        pltpu.SemaphoreType.DMA,
    ],
)(_sc_body)


def _merge_body(p_ref, o_ref):
    o_ref[...] = p_ref[0] + p_ref[1]


def kernel(ray_samples_packed, rgb_samples, weights_samples):
    ids = ray_samples_packed.astype(jnp.int32)
    rgb_flat = rgb_samples.reshape(-1)
    w_flat = weights_samples.reshape(-1)
    zeros = jnp.zeros((3 * R,), jnp.float32)
    partial = _sc_scatter(ids, rgb_flat, w_flat, zeros)
    out = pl.pallas_call(
        _merge_body,
        out_shape=jax.ShapeDtypeStruct((R * 3 // 128, 128), jnp.float32),
    )(partial.reshape(NC, R * 3 // 128, 128))
    return out.reshape(R, 3)


# speedup vs baseline: 26.1647x; 12.4985x over previous
import functools

import jax
import jax.numpy as jnp
from jax import lax
from jax.experimental import pallas as pl
from jax.experimental.pallas import tpu as pltpu
from jax.experimental.pallas import tpu_sc as plsc

N = 4194304
R = 65536
NC = 2
NS = 16
W = NC * NS
C = N // W
CH = 2048
NCH = C // CH
NJ = CH // 128
RT = R // NS


def _sc_body(ids_hbm, r_hbm, g_hbm, b_hbm, w_hbm, zeros_hbm, out_hbm,
             ids_v, r_v, g_v, b_v, w_v, cr_v, cg_v, cb_v,
             acc_r, acc_g, acc_b, sem_in, sem_sc, sem_z):
    cid = lax.axis_index("c")
    sid = lax.axis_index("s")
    wid = cid * NS + sid

    for acc in (acc_r, acc_g, acc_b):
        pltpu.async_copy(zeros_hbm.at[pl.ds(sid * RT, RT)],
                         acc.at[pl.ds(sid * RT, RT)], sem_z).wait()
    plsc.subcore_barrier()

    planes = ((r_hbm, r_v, cr_v, acc_r),
              (g_hbm, g_v, cg_v, acc_g),
              (b_hbm, b_v, cb_v, acc_b))

    def in_copies(p, ch):
        base = wid * C + ch * CH
        copies = [
            pltpu.make_async_copy(ids_hbm.at[pl.ds(base + 128 * j, 128)],
                                  ids_v[p].at[j], sem_in[p])
            for j in range(NJ)
        ]
        copies.append(pltpu.make_async_copy(w_hbm.at[pl.ds(base, CH)],
                                            w_v[p], sem_in[p]))
        copies.extend(
            pltpu.make_async_copy(x_hbm.at[pl.ds(base, CH)],
                                  x_v[p], sem_in[p])
            for x_hbm, x_v, _, _ in planes)
        return copies

    def issue_in(p, ch):
        for c in in_copies(p, ch):
            c.start()

    def wait_in(p, ch):
        for c in in_copies(p, ch):
            c.wait()

    def compute(p):
        def group(j, _):
            for gg in range(8):
                o = 128 * j + 16 * gg
                w16 = w_v[p][pl.ds(o, 16)]
                for _, x_v, c_v, _ in planes:
                    c_v[p][pl.ds(o, 16)] = x_v[p][pl.ds(o, 16)] * w16
            return 0

        lax.fori_loop(0, NJ, group, 0)

    def fire_scatters(p):
        for j in range(NJ):
            idx = ids_v[p].at[j]
            for _, _, c_v, acc in planes:
                pltpu.async_copy(c_v[p].at[pl.ds(128 * j, 128)],
                                 acc.at[idx], sem_sc[p], add=True)

    def drain_scatters(p):
        for _ in range(3):
            pltpu.make_async_copy(w_hbm.at[pl.ds(0, CH)],
                                  w_v[p], sem_sc[p]).wait()

    issue_in(0, 0)

    def pair(ch2, _):
        ch0 = 2 * ch2

        @pl.when(ch2 > 0)
        def _():
            drain_scatters(0)
        issue_in(1, ch0 + 1)
        wait_in(0, ch0)
        compute(0)
        fire_scatters(0)

        @pl.when(ch2 > 0)
        def _():
            drain_scatters(1)

        @pl.when(ch2 < NCH // 2 - 1)
        def _():
            issue_in(0, ch0 + 2)
        wait_in(1, ch0 + 1)
        compute(1)
        fire_scatters(1)
        return 0

    lax.fori_loop(0, NCH // 2, pair, 0)
    drain_scatters(0)
    drain_scatters(1)
    plsc.subcore_barrier()

    for x, (_, _, _, acc) in enumerate(planes):
        pltpu.async_copy(acc.at[pl.ds(sid * RT, RT)],
                         out_hbm.at[cid].at[pl.ds(x * R + sid * RT, RT)],
                         sem_z).wait()


_sc_scatter = functools.partial(
    pl.kernel,
    mesh=plsc.VectorSubcoreMesh(core_axis_name="c", subcore_axis_name="s",
                                num_cores=NC, num_subcores=NS),
    compiler_params=pltpu.CompilerParams(needs_layout_passes=False),
    out_type=jax.ShapeDtypeStruct((NC, 3 * R), jnp.float32),
    scratch_types=[
        [pltpu.VMEM((NJ, 128), jnp.int32)] * 2,
        [pltpu.VMEM((CH,), jnp.float32)] * 2,
        [pltpu.VMEM((CH,), jnp.float32)] * 2,
        [pltpu.VMEM((CH,), jnp.float32)] * 2,
        [pltpu.VMEM((CH,), jnp.float32)] * 2,
        [pltpu.VMEM((CH,), jnp.float32)] * 2,
        [pltpu.VMEM((CH,), jnp.float32)] * 2,
        [pltpu.VMEM((CH,), jnp.float32)] * 2,
        pltpu.VMEM_SHARED((R,), jnp.float32),
        pltpu.VMEM_SHARED((R,), jnp.float32),
        pltpu.VMEM_SHARED((R,), jnp.float32),
        [pltpu.SemaphoreType.DMA] * 2,
        [pltpu.SemaphoreType.DMA] * 2,
        pltpu.SemaphoreType.DMA,
    ],
)(_sc_body)


def _merge_body(p_ref, o_ref):
    o_ref[...] = (p_ref[0] + p_ref[1]).T


def kernel(ray_samples_packed, rgb_samples, weights_samples):
    zeros = jnp.zeros((R,), jnp.float32)
    partial = _sc_scatter(ray_samples_packed,
                          rgb_samples[:, 0], rgb_samples[:, 1],
                          rgb_samples[:, 2], weights_samples[:, 0], zeros)
    return pl.pallas_call(
        _merge_body,
        out_shape=jax.ShapeDtypeStruct((R, 3), jnp.float32),
    )(partial.reshape(NC, 3, R))
